# baseline (speedup 1.0000x reference)
import functools

import jax
import jax.numpy as jnp
from jax import lax
from jax.experimental import pallas as pl
from jax.experimental.pallas import tpu as pltpu
from jax.experimental.pallas import tpu_sc as plsc

D = 64
FH, FL = 8, 8
BL = 128
TW = BL + 1
NC, NS = 2, 16
NW = NC * NS
NGB = 4
NTB = 2


@functools.cache
def _build_relayout(V):
    NFB = (V - D) // BL
    TAILROWS = D // 2
    SCR = NFB * (BL // 2) + TAILROWS + (BL // 4)
    assert (V - D) % BL == 0
    per, extra = divmod(NFB, NW)

    mesh = plsc.VectorSubcoreMesh(core_axis_name="c", subcore_axis_name="s")

    @functools.partial(
        pl.kernel,
        out_type=jax.ShapeDtypeStruct((SCR, BL), jnp.float32),
        mesh=mesh,
        scratch_types=[
            [pltpu.VMEM((D, TW), jnp.float32)] * 2,
            [pltpu.VMEM((D, BL), jnp.float32)] * 2,
            [pltpu.SemaphoreType.DMA] * 2,
            [pltpu.SemaphoreType.DMA] * 2,
        ],
        compiler_params=pltpu.CompilerParams(
            use_tc_tiling_on_sc=True, needs_layout_passes=False),
    )
    def relayout(tT_hbm, tail_hbm, scr_hbm, tilebuf, obuf, tsem, wsem):
        wid = lax.axis_index("s") * NC + lax.axis_index("c")
        cnt = per + (wid < extra)
        base = wid * per + jnp.minimum(wid, extra)

        def tile_desc(n, b, fh):
            ihi = base + n
            return pltpu.make_async_copy(
                tT_hbm.at[pl.ds(fh * FL, FL), pl.ds(ihi * BL, BL)],
                tilebuf[b].at[pl.ds(fh * FL, FL), pl.ds(0, BL)],
                tsem[b])

        iota16 = lax.iota(jnp.int32, 16)
        rows = [f0 * 16 + iota16 for f0 in range(D // 16)]

        def transpose(b):
            src = tilebuf[b]
            dst = obuf[b]

            @pl.loop(0, BL // 2)
            def _p(p):
                for h in range(2):
                    col = jnp.full((16,), 2 * p + h, jnp.int32)
                    vs = [plsc.load_gather(src, [rows[f0], col])
                          for f0 in range(D // 16)]
                    for f0 in range(D // 16):
                        dst[p, pl.ds(h * D + f0 * 16, 16)] = vs[f0]

        def fire_write(n, b):
            pltpu.make_async_copy(
                obuf[b],
                scr_hbm.at[pl.ds((base + n) * (BL // 2), BL // 2)],
                wsem[b]).start()

        def drain_write(b):
            pltpu.make_async_copy(
                scr_hbm.at[pl.ds(0, BL // 2)], obuf[b], wsem[b]).wait()

        for fh in range(FH):
            tile_desc(0, 0, fh).start()

        @pl.loop(0, per + 2, step=2)
        def _n(n0):
            for b in range(2):
                n = n0 + b

                @pl.when(n + 1 < cnt)
                def _():
                    for fh in range(FH):
                        tile_desc(n + 1, 1 - b, fh).start()

                @pl.when(n < cnt)
                def _():
                    for fh in range(FH):
                        tile_desc(n, b, fh).wait()

                    @pl.when(n >= 2)
                    def _():
                        drain_write(b)

                    transpose(b)
                    fire_write(n, b)

        drain_write(0)
        drain_write(1)

        @pl.when(wid == NW - 1)
        def _():
            pltpu.sync_copy(tail_hbm,
                            tilebuf[0].at[pl.ds(0, TAILROWS), pl.ds(0, BL)])
            pltpu.sync_copy(tilebuf[0].at[pl.ds(0, TAILROWS), pl.ds(0, BL)],
                            scr_hbm.at[pl.ds(NFB * (BL // 2), TAILROWS)])

    return relayout


@functools.cache
def _build_gather(NB, NT, V2):
    NBB = NB // BL
    BBW = NBB // NW
    UNITS = NT * BBW
    assert NB % (BL * NW) == 0 and UNITS % NGB == 0

    mesh = plsc.VectorSubcoreMesh(core_axis_name="c", subcore_axis_name="s")

    @functools.partial(
        pl.kernel,
        out_type=jax.ShapeDtypeStruct((NT, FH, NBB, FL, BL), jnp.float32),
        mesh=mesh,
        scratch_types=[
            pltpu.VMEM((NT, BBW * BL), jnp.int32),
            [pltpu.VMEM((BL, D), jnp.float32)] * NGB,
            [pltpu.VMEM((FH, FL, TW), jnp.float32)] * NTB,
            [pltpu.SemaphoreType.DMA] * NGB,
            [pltpu.SemaphoreType.DMA] * NTB,
        ],
        compiler_params=pltpu.CompilerParams(
            use_tc_tiling_on_sc=False, needs_layout_passes=False),
    )
    def emb(xT_hbm, table_hbm, out_hbm, idx_v, gbuf, tbuf, gsem, wsem):
        wid = lax.axis_index("s") * NC + lax.axis_index("c")
        col0 = wid * (BBW * BL)
        pltpu.sync_copy(xT_hbm.at[:, pl.ds(col0, BBW * BL)], idx_v)

        def coords(u):
            return u // BBW, u % BBW

        def gather_desc(u, g):
            t, j = coords(u)
            return pltpu.make_async_copy(
                table_hbm.at[idx_v.at[t, pl.ds(j * BL, BL)]],
                gbuf[g], gsem[g])

        iota16 = lax.iota(jnp.int32, 16)
        rows_hi = [(f0 * 16 + iota16) // FL for f0 in range(D // 16)]
        rows_lo = [(f0 * 16 + iota16) % FL for f0 in range(D // 16)]

        def transpose(g, w):
            src = gbuf[g]
            dst = tbuf[w]

            @pl.loop(0, BL // 16)
            def _jo(j0):
                for jj in range(16):
                    j = j0 * 16 + jj
                    col = jnp.full((16,), j, jnp.int32)
                    vs = [src[j, pl.ds(f0 * 16, 16)] for f0 in range(D // 16)]
                    for f0 in range(D // 16):
                        plsc.store_scatter(
                            dst, [rows_hi[f0], rows_lo[f0], col], vs[f0])

        def fire_write(u, w):
            t, j = coords(u)
            bb = wid * BBW + j
            pltpu.make_async_copy(
                tbuf[w].at[:, :, pl.ds(0, BL)],
                out_hbm.at[t, :, bb, :, :], wsem[w]).start()

        def drain_write(w):
            pltpu.make_async_copy(
                out_hbm.at[0, :, 0, :, :],
                tbuf[w].at[:, :, pl.ds(0, BL)], wsem[w]).wait()

        for u in range(NGB - 1):
            gather_desc(u, u).start()

        @pl.loop(0, UNITS, step=NGB)
        def _u(u0):
            for b in range(NGB):
                u = u0 + b
                w = b % NTB

                @pl.when(u + (NGB - 1) < UNITS)
                def _():
                    gather_desc(u + (NGB - 1), (b + NGB - 1) % NGB).start()

                gather_desc(u, b).wait()

                @pl.when(u >= NTB)
                def _():
                    drain_write(w)

                transpose(b, w)
                fire_write(u, w)

        drain_write(0)
        drain_write(1)

    return emb


def kernel(x, table):
    NB, NT = x.shape
    V = table.shape[0]
    tail = table[V - D:].reshape(D // 2, BL)
    scr = _build_relayout(V)(table.T, tail)
    t64 = scr.reshape(scr.shape[0] * 2, D)
    xT = x.T.astype(jnp.int32)
    out5 = _build_gather(NB, NT, t64.shape[0])(xT, t64)
    return out5.transpose(2, 4, 0, 1, 3).reshape(NB, NT, D)


# speedup vs baseline: 1.5817x; 1.5817x over previous
import functools

import jax
import jax.numpy as jnp
from jax import lax
from jax.experimental import pallas as pl
from jax.experimental.pallas import tpu as pltpu
from jax.experimental.pallas import tpu_sc as plsc

D = 64
FH, FL = 8, 8
BL = 128
TW = BL + 1
NC, NS = 2, 16
NW = NC * NS
NGB = 4
NTB = 2


@functools.cache
def _build_tiledump(V):
    NFB = (V - D) // BL
    assert (V - D) % BL == 0
    per, extra = divmod(NFB, NW)

    mesh = plsc.VectorSubcoreMesh(core_axis_name="c", subcore_axis_name="s")

    @functools.partial(
        pl.kernel,
        out_type=jax.ShapeDtypeStruct((NFB, D, BL), jnp.float32),
        mesh=mesh,
        scratch_types=[
            [pltpu.VMEM((D, BL), jnp.float32)] * 2,
            [pltpu.SemaphoreType.DMA] * 2,
            [pltpu.SemaphoreType.DMA] * 2,
        ],
        compiler_params=pltpu.CompilerParams(
            use_tc_tiling_on_sc=True, needs_layout_passes=False),
    )
    def dump(tT_hbm, td_hbm, buf, tsem, wsem):
        wid = lax.axis_index("s") * NC + lax.axis_index("c")
        cnt = per + (wid < extra)
        base = wid * per + jnp.minimum(wid, extra)

        def tile_desc(n, b, fh):
            ihi = base + n
            return pltpu.make_async_copy(
                tT_hbm.at[pl.ds(fh * FL, FL), pl.ds(ihi * BL, BL)],
                buf[b].at[pl.ds(fh * FL, FL), pl.ds(0, BL)],
                tsem[b])

        def write_desc(n, b):
            return pltpu.make_async_copy(
                buf[b], td_hbm.at[base + n], wsem[b])

        for fh in range(FH):
            tile_desc(0, 0, fh).start()

        @pl.loop(0, per + 2, step=2)
        def _n(n0):
            for b in range(2):
                n = n0 + b

                @pl.when(n + 1 < cnt)
                def _():
                    @pl.when(n >= 1)
                    def _():
                        write_desc(n - 1, 1 - b).wait()

                    for fh in range(FH):
                        tile_desc(n + 1, 1 - b, fh).start()

                @pl.when(n < cnt)
                def _():
                    for fh in range(FH):
                        tile_desc(n, b, fh).wait()

                    write_desc(n, b).start()

        write_desc(0, 0).wait()
        write_desc(0, 1).wait()

    return dump


@functools.cache
def _build_pairs(V):
    NFB = (V - D) // BL
    TAILROWS = D // 2
    SCR = NFB * (BL // 2) + TAILROWS + (BL // 4)
    per, extra = divmod(NFB, NW)

    mesh = plsc.VectorSubcoreMesh(core_axis_name="c", subcore_axis_name="s")

    @functools.partial(
        pl.kernel,
        out_type=jax.ShapeDtypeStruct((SCR, BL), jnp.float32),
        mesh=mesh,
        scratch_types=[
            [pltpu.VMEM((D, TW), jnp.float32)] * 2,
            [pltpu.VMEM((D, BL), jnp.float32)] * 2,
            [pltpu.SemaphoreType.DMA] * 2,
            [pltpu.SemaphoreType.DMA] * 2,
        ],
        compiler_params=pltpu.CompilerParams(
            use_tc_tiling_on_sc=False, needs_layout_passes=False),
    )
    def pairs(td_hbm, tail_hbm, scr_hbm, tilebuf, obuf, tsem, wsem):
        wid = lax.axis_index("s") * NC + lax.axis_index("c")
        cnt = per + (wid < extra)
        base = wid * per + jnp.minimum(wid, extra)

        def tile_desc(n, b):
            return pltpu.make_async_copy(
                td_hbm.at[base + n],
                tilebuf[b].at[pl.ds(0, D), pl.ds(0, BL)],
                tsem[b])

        iota16 = lax.iota(jnp.int32, 16)
        rows = [f0 * 16 + iota16 for f0 in range(D // 16)]

        def transpose(b):
            src = tilebuf[b]
            dst = obuf[b]

            @pl.loop(0, BL // 2)
            def _p(p):
                for h in range(2):
                    col = jnp.full((16,), 2 * p + h, jnp.int32)
                    vs = [plsc.load_gather(src, [rows[f0], col])
                          for f0 in range(D // 16)]
                    for f0 in range(D // 16):
                        dst[p, pl.ds(h * D + f0 * 16, 16)] = vs[f0]

        def fire_write(n, b):
            pltpu.make_async_copy(
                obuf[b],
                scr_hbm.at[pl.ds((base + n) * (BL // 2), BL // 2)],
                wsem[b]).start()

        def drain_write(b):
            pltpu.make_async_copy(
                scr_hbm.at[pl.ds(0, BL // 2)], obuf[b], wsem[b]).wait()

        tile_desc(0, 0).start()

        @pl.loop(0, per + 2, step=2)
        def _n(n0):
            for b in range(2):
                n = n0 + b

                @pl.when(n + 1 < cnt)
                def _():
                    tile_desc(n + 1, 1 - b).start()

                @pl.when(n < cnt)
                def _():
                    tile_desc(n, b).wait()

                    @pl.when(n >= 2)
                    def _():
                        drain_write(b)

                    transpose(b)
                    fire_write(n, b)

        drain_write(0)
        drain_write(1)

        @pl.when(wid == NW - 1)
        def _():
            pltpu.sync_copy(tail_hbm,
                            tilebuf[0].at[pl.ds(0, TAILROWS), pl.ds(0, BL)])
            pltpu.sync_copy(tilebuf[0].at[pl.ds(0, TAILROWS), pl.ds(0, BL)],
                            scr_hbm.at[pl.ds(NFB * (BL // 2), TAILROWS)])

    return pairs


@functools.cache
def _build_gather(NB, NT, V2):
    NBB = NB // BL
    BBW = NBB // NW
    UNITS = NT * BBW
    assert NB % (BL * NW) == 0 and UNITS % NGB == 0

    mesh = plsc.VectorSubcoreMesh(core_axis_name="c", subcore_axis_name="s")

    @functools.partial(
        pl.kernel,
        out_type=jax.ShapeDtypeStruct((NT, FH, NBB, FL, BL), jnp.float32),
        mesh=mesh,
        scratch_types=[
            pltpu.VMEM((NT, BBW * BL), jnp.int32),
            [pltpu.VMEM((BL, D), jnp.float32)] * NGB,
            [pltpu.VMEM((FH, FL, TW), jnp.float32)] * NTB,
            [pltpu.SemaphoreType.DMA] * NGB,
            [pltpu.SemaphoreType.DMA] * NTB,
        ],
        compiler_params=pltpu.CompilerParams(
            use_tc_tiling_on_sc=False, needs_layout_passes=False),
    )
    def emb(xT_hbm, table_hbm, out_hbm, idx_v, gbuf, tbuf, gsem, wsem):
        wid = lax.axis_index("s") * NC + lax.axis_index("c")
        col0 = wid * (BBW * BL)
        pltpu.sync_copy(xT_hbm.at[:, pl.ds(col0, BBW * BL)], idx_v)

        def coords(u):
            return u // BBW, u % BBW

        def gather_desc(u, g):
            t, j = coords(u)
            return pltpu.make_async_copy(
                table_hbm.at[idx_v.at[t, pl.ds(j * BL, BL)]],
                gbuf[g], gsem[g])

        iota16 = lax.iota(jnp.int32, 16)
        rows_hi = [(f0 * 16 + iota16) // FL for f0 in range(D // 16)]
        rows_lo = [(f0 * 16 + iota16) % FL for f0 in range(D // 16)]

        def transpose(g, w):
            src = gbuf[g]
            dst = tbuf[w]

            @pl.loop(0, BL // 16)
            def _jo(j0):
                for jj in range(16):
                    j = j0 * 16 + jj
                    col = jnp.full((16,), j, jnp.int32)
                    vs = [src[j, pl.ds(f0 * 16, 16)] for f0 in range(D // 16)]
                    for f0 in range(D // 16):
                        plsc.store_scatter(
                            dst, [rows_hi[f0], rows_lo[f0], col], vs[f0])

        def fire_write(u, w):
            t, j = coords(u)
            bb = wid * BBW + j
            pltpu.make_async_copy(
                tbuf[w].at[:, :, pl.ds(0, BL)],
                out_hbm.at[t, :, bb, :, :], wsem[w]).start()

        def drain_write(w):
            pltpu.make_async_copy(
                out_hbm.at[0, :, 0, :, :],
                tbuf[w].at[:, :, pl.ds(0, BL)], wsem[w]).wait()

        for u in range(NGB - 1):
            gather_desc(u, u).start()

        @pl.loop(0, UNITS, step=NGB)
        def _u(u0):
            for b in range(NGB):
                u = u0 + b
                w = b % NTB

                @pl.when(u + (NGB - 1) < UNITS)
                def _():
                    gather_desc(u + (NGB - 1), (b + NGB - 1) % NGB).start()

                gather_desc(u, b).wait()

                @pl.when(u >= NTB)
                def _():
                    drain_write(w)

                transpose(b, w)
                fire_write(u, w)

        drain_write(0)
        drain_write(1)

    return emb


def kernel(x, table):
    NB, NT = x.shape
    V = table.shape[0]
    tail = table[V - D:].reshape(D // 2, BL)
    td = _build_tiledump(V)(table.T)
    scr = _build_pairs(V)(td, tail)
    t64 = scr.reshape(scr.shape[0] * 2, D)
    xT = x.T.astype(jnp.int32)
    out5 = _build_gather(NB, NT, t64.shape[0])(xT, t64)
    return out5.transpose(2, 4, 0, 1, 3).reshape(NB, NT, D)


# speedup vs baseline: 1.8376x; 1.1618x over previous
import functools

import jax
import jax.numpy as jnp
from jax import lax
from jax.experimental import pallas as pl
from jax.experimental.pallas import tpu as pltpu
from jax.experimental.pallas import tpu_sc as plsc

D = 64
FH, FL = 8, 8
BL = 128
TW = BL + 1
NC, NS = 2, 16
NW = NC * NS
NGB = 4
NTB = 2


@functools.cache
def _build(NB, NT):
    NBB = NB // BL
    BBW = NBB // NW
    UNITS = NT * BBW
    assert NB % (BL * NW) == 0 and UNITS % NGB == 0

    mesh = plsc.VectorSubcoreMesh(core_axis_name="c", subcore_axis_name="s")

    @functools.partial(
        pl.kernel,
        out_type=jax.ShapeDtypeStruct((NT, FH, NBB, FL, BL), jnp.float32),
        mesh=mesh,
        scratch_types=[
            pltpu.VMEM((NT, BBW * BL), jnp.int32),
            [pltpu.VMEM((BL, D), jnp.float32)] * NGB,
            [pltpu.VMEM((FH, FL, TW), jnp.float32)] * NTB,
            [pltpu.SemaphoreType.DMA] * NGB,
            [pltpu.SemaphoreType.DMA] * NTB,
        ],
        compiler_params=pltpu.CompilerParams(
            use_tc_tiling_on_sc=False, needs_layout_passes=False),
    )
    def emb(xT_hbm, table_hbm, out_hbm, idx_v, gbuf, tbuf, gsem, wsem):
        wid = lax.axis_index("s") * NC + lax.axis_index("c")
        col0 = wid * (BBW * BL)
        pltpu.sync_copy(xT_hbm.at[:, pl.ds(col0, BBW * BL)], idx_v)

        def coords(u):
            return u // BBW, u % BBW

        def gather_desc(u, g):
            t, j = coords(u)
            return pltpu.make_async_copy(
                table_hbm.at[idx_v.at[t, pl.ds(j * BL, BL)]],
                gbuf[g], gsem[g])

        iota16 = lax.iota(jnp.int32, 16)
        rows_hi = [(f0 * 16 + iota16) // FL for f0 in range(D // 16)]
        rows_lo = [(f0 * 16 + iota16) % FL for f0 in range(D // 16)]

        def transpose(g, w):
            src = gbuf[g]
            dst = tbuf[w]

            @pl.loop(0, BL // 16)
            def _jo(j0):
                prev = prev_col = None
                for jj in range(17):
                    cur = cur_col = None
                    if jj < 16:
                        j = j0 * 16 + jj
                        cur_col = jnp.full((16,), j, jnp.int32)
                        cur = [src[j, pl.ds(f0 * 16, 16)]
                               for f0 in range(D // 16)]
                    if prev is not None:
                        for f0 in range(D // 16):
                            plsc.store_scatter(
                                dst, [rows_hi[f0], rows_lo[f0], prev_col],
                                prev[f0])
                    prev, prev_col = cur, cur_col

        def fire_write(u, w):
            t, j = coords(u)
            bb = wid * BBW + j
            pltpu.make_async_copy(
                tbuf[w].at[:, :, pl.ds(0, BL)],
                out_hbm.at[t, :, bb, :, :], wsem[w]).start()

        def drain_write(w):
            pltpu.make_async_copy(
                out_hbm.at[0, :, 0, :, :],
                tbuf[w].at[:, :, pl.ds(0, BL)], wsem[w]).wait()

        for u in range(NGB - 1):
            gather_desc(u, u).start()

        @pl.loop(0, UNITS, step=NGB)
        def _u(u0):
            for b in range(NGB):
                u = u0 + b
                w = b % NTB

                @pl.when(u + (NGB - 1) < UNITS)
                def _():
                    gather_desc(u + (NGB - 1), (b + NGB - 1) % NGB).start()

                gather_desc(u, b).wait()

                @pl.when(u >= NTB)
                def _():
                    drain_write(w)

                transpose(b, w)
                fire_write(u, w)

        drain_write(0)
        drain_write(1)

    return emb


def kernel(x, table):
    NB, NT = x.shape
    xT = x.T.astype(jnp.int32)
    out5 = _build(NB, NT)(xT, table)
    return out5.transpose(2, 4, 0, 1, 3).reshape(NB, NT, D)
